# Initial kernel scaffold; baseline (speedup 1.0000x reference)
import jax
import jax.numpy as jnp
from jax.experimental import pallas as pl


def kernel(yhat, y):
    raise NotImplementedError("write your pallas kernel here")


# speedup vs baseline: 1.0727x; 1.0727x over previous
import functools

import jax
import jax.numpy as jnp
from jax import lax
from jax.experimental import pallas as pl
from jax.experimental.pallas import tpu as pltpu
from jax.experimental.pallas import tpu_sc as plsc

TOPK = 5
NROWS = 128
N = 32768
NC = 2
NS = 16
NW = NC * NS
ROWS_PER_W = NROWS // NW
L = 16
NV = N // L


def _sc_body(yhat_hbm, y_hbm, out_hbm, ybuf, hbuf, obuf, sem_y, sem_h):
  wid = lax.axis_index("s") * NC + lax.axis_index("c")
  iota = lax.iota(jnp.int32, L)
  hits = jnp.float32(0.0)
  for i in range(ROWS_PER_W):
    r = wid * ROWS_PER_W + i
    cp_h = pltpu.make_async_copy(yhat_hbm.at[r], hbuf, sem_h)
    cp_h.start()
    cp_y = pltpu.make_async_copy(y_hbm.at[r], ybuf, sem_y)
    cp_y.start()
    cp_y.wait()

    def amax_body(j, c):
      bv, bi = c
      x = ybuf[pl.ds(j * L, L)]
      idx = iota + j * L
      p = x > bv
      return jnp.where(p, x, bv), jnp.where(p, idx, bi)

    bv, bi = lax.fori_loop(
        0, NV, amax_body,
        (jnp.full((L,), jnp.finfo(jnp.float32).min, jnp.float32),
         jnp.zeros((L,), jnp.int32)))
    m = jnp.max(bv)
    t = jnp.min(jnp.where(bv == m, bi, jnp.int32(N)))

    cp_h.wait()
    tvec = jnp.full((L,), t, jnp.int32)
    v = plsc.load_gather(hbuf, [tvec])

    def cnt_body(j, c):
      x = hbuf[pl.ds(j * L, L)]
      idx = iota + j * L
      gt = (x > v).astype(jnp.int32)
      eqb = ((x == v) & (idx < t)).astype(jnp.int32)
      return c + gt + eqb

    cnt = lax.fori_loop(0, NV, cnt_body, jnp.zeros((L,), jnp.int32))
    rank = jnp.sum(cnt)
    hits = hits + jnp.where(rank < TOPK, jnp.float32(1.0), jnp.float32(0.0))

  obuf[...] = jnp.full((L,), hits, jnp.float32)
  pltpu.sync_copy(obuf, out_hbm.at[wid])


def _tc_merge(p_ref, o_ref):
  total = jnp.sum(p_ref[...]) * (1.0 / L)
  o_ref[...] = jnp.full((1, 1), (1.0 - total / NROWS) * 100.0, jnp.float32)


@jax.jit
def kernel(yhat, y):
  y2d = jnp.reshape(y, (NROWS, N))

  mesh = plsc.VectorSubcoreMesh(core_axis_name="c", subcore_axis_name="s")
  sc_k = functools.partial(
      pl.kernel,
      mesh=mesh,
      compiler_params=pltpu.CompilerParams(needs_layout_passes=False),
      out_type=jax.ShapeDtypeStruct((NW, L), jnp.float32),
      scratch_types=[
          pltpu.VMEM((N,), jnp.float32),
          pltpu.VMEM((N,), jnp.float32),
          pltpu.VMEM((L,), jnp.float32),
          pltpu.SemaphoreType.DMA,
          pltpu.SemaphoreType.DMA,
      ],
  )(_sc_body)
  partial_hits = sc_k(yhat, y2d)

  err = pl.pallas_call(
      _tc_merge,
      out_shape=jax.ShapeDtypeStruct((1, 1), jnp.float32),
  )(partial_hits)
  return jnp.reshape(err, ())


# speedup vs baseline: 1.5914x; 1.4835x over previous
import functools

import jax
import jax.numpy as jnp
from jax import lax
from jax.experimental import pallas as pl
from jax.experimental.pallas import tpu as pltpu
from jax.experimental.pallas import tpu_sc as plsc

TOPK = 5
NROWS = 128
N = 32768
NC = 2
NS = 16
NW = NC * NS
ROWS_PER_W = NROWS // NW
L = 16
U = 8
CH = L * U
NIT = N // CH
NPH = 2 * ROWS_PER_W
F32_MIN = jnp.finfo(jnp.float32).min


def _sc_body(yhat_hbm, y_hbm, out_hbm, buf0, buf1, buf2, obuf,
             sem0, sem1, sem2):
  bufs = (buf0, buf1, buf2)
  sems = (sem0, sem1, sem2)
  wid = lax.axis_index("s") * NC + lax.axis_index("c")
  base_row = wid * ROWS_PER_W
  iota = lax.iota(jnp.int32, L)
  iotas = [iota + u * L for u in range(U)]

  def copy(p):
    src = (y_hbm if p % 2 == 0 else yhat_hbm).at[base_row + p // 2]
    return pltpu.make_async_copy(src, bufs[p % 3], sems[p % 3])

  copy(0).start()
  copy(1).start()
  hits = jnp.float32(0.0)
  t = jnp.int32(0)
  for p in range(NPH):
    if p + 2 < NPH:
      copy(p + 2).start()
    copy(p).wait()
    buf = bufs[p % 3]

    if p % 2 == 0:
      def amax_body(j, c, buf=buf):
        bvs, bis = c[:U], c[U:]
        base = j * CH
        nbvs, nbis = [], []
        for u in range(U):
          x = buf[pl.ds(base + u * L, L)]
          pgt = x > bvs[u]
          nbvs.append(jnp.where(pgt, x, bvs[u]))
          nbis.append(jnp.where(pgt, iotas[u] + base, bis[u]))
        return tuple(nbvs) + tuple(nbis)

      c = lax.fori_loop(
          0, NIT, amax_body,
          tuple(jnp.full((L,), F32_MIN, jnp.float32) for _ in range(U))
          + tuple(jnp.zeros((L,), jnp.int32) for _ in range(U)))
      bvs, bis = c[:U], c[U:]
      m = jnp.max(bvs[0])
      for u in range(1, U):
        m = jnp.maximum(m, jnp.max(bvs[u]))
      t = jnp.int32(N)
      for u in range(U):
        t = jnp.minimum(t, jnp.min(jnp.where(bvs[u] == m, bis[u],
                                             jnp.int32(N))))
    else:
      v = plsc.load_gather(buf, [jnp.full((L,), t, jnp.int32)])

      def cnt_body(j, c, buf=buf, v=v, t=t):
        base = j * CH
        out = []
        for u in range(U):
          x = buf[pl.ds(base + u * L, L)]
          gt = (x > v).astype(jnp.int32)
          eqb = ((x == v) & (iotas[u] + base < t)).astype(jnp.int32)
          out.append(c[u] + gt + eqb)
        return tuple(out)

      c = lax.fori_loop(0, NIT, cnt_body,
                        tuple(jnp.zeros((L,), jnp.int32) for _ in range(U)))
      rank = jnp.sum(c[0])
      for u in range(1, U):
        rank = rank + jnp.sum(c[u])
      hits = hits + jnp.where(rank < TOPK, jnp.float32(1.0), jnp.float32(0.0))

  obuf[...] = jnp.full((L,), hits, jnp.float32)
  pltpu.sync_copy(obuf, out_hbm.at[wid])


def _tc_merge(p_ref, o_ref):
  total = jnp.sum(p_ref[...]) * (1.0 / L)
  o_ref[...] = jnp.full((1, 1), (1.0 - total / NROWS) * 100.0, jnp.float32)


@jax.jit
def kernel(yhat, y):
  y2d = jnp.reshape(y, (NROWS, N))

  mesh = plsc.VectorSubcoreMesh(core_axis_name="c", subcore_axis_name="s")
  sc_k = functools.partial(
      pl.kernel,
      mesh=mesh,
      compiler_params=pltpu.CompilerParams(needs_layout_passes=False),
      out_type=jax.ShapeDtypeStruct((NW, L), jnp.float32),
      scratch_types=[
          pltpu.VMEM((N,), jnp.float32),
          pltpu.VMEM((N,), jnp.float32),
          pltpu.VMEM((N,), jnp.float32),
          pltpu.VMEM((L,), jnp.float32),
          pltpu.SemaphoreType.DMA,
          pltpu.SemaphoreType.DMA,
          pltpu.SemaphoreType.DMA,
      ],
  )(_sc_body)
  partial_hits = sc_k(yhat, y2d)

  err = pl.pallas_call(
      _tc_merge,
      out_shape=jax.ShapeDtypeStruct((1, 1), jnp.float32),
  )(partial_hits)
  return jnp.reshape(err, ())


# speedup vs baseline: 2.1385x; 1.3438x over previous
import functools

import jax
import jax.numpy as jnp
from jax import lax
from jax.experimental import pallas as pl
from jax.experimental.pallas import tpu as pltpu
from jax.experimental.pallas import tpu_sc as plsc

TOPK = 5
NROWS = 128
N = 32768
NC = 2
NS = 16
NW = NC * NS
ROWS_PER_W = NROWS // NW
L = 16
U = 8
CH = L * U
NIT = N // CH
CBLK = 2048
NBLK = N // CBLK
F32_MIN = jnp.finfo(jnp.float32).min


def _tc_argmax(y_ref, o_ref, mx_ref, ix_ref):
  j = pl.program_id(0)
  x = y_ref[...]
  cols = lax.broadcasted_iota(jnp.int32, (NROWS, CBLK), 1) + j * CBLK
  m = jnp.max(x, axis=1, keepdims=True)
  idx = jnp.min(jnp.where(x == m, cols, jnp.int32(N)), axis=1, keepdims=True)

  @pl.when(j == 0)
  def _():
    mx_ref[...] = m
    ix_ref[...] = idx

  @pl.when(j > 0)
  def _():
    upd = m > mx_ref[...]
    mx_ref[...] = jnp.where(upd, m, mx_ref[...])
    ix_ref[...] = jnp.where(upd, idx, ix_ref[...])

  @pl.when(j == NBLK - 1)
  def _():
    o_ref[...] = jnp.reshape(ix_ref[...], (NROWS // L, L))


def _sc_body(yhat_hbm, t_hbm, out_hbm, buf0, buf1, tbuf, obuf,
             sem0, sem1, semt):
  bufs = (buf0, buf1)
  sems = (sem0, sem1)
  wid = lax.axis_index("s") * NC + lax.axis_index("c")
  base_row = wid * ROWS_PER_W
  iota = lax.iota(jnp.int32, L)
  iotas = [iota + u * L for u in range(U)]

  pltpu.make_async_copy(t_hbm, tbuf, semt).start()

  def copy(i):
    return pltpu.make_async_copy(yhat_hbm.at[base_row + i], bufs[i % 2],
                                 sems[i % 2])

  copy(0).start()
  pltpu.make_async_copy(t_hbm, tbuf, semt).wait()

  hits = jnp.float32(0.0)
  for i in range(ROWS_PER_W):
    if i + 1 < ROWS_PER_W:
      copy(i + 1).start()
    copy(i).wait()
    buf = bufs[i % 2]

    r = base_row + i
    tvec = plsc.load_gather(
        tbuf, [jnp.full((L,), r // L, jnp.int32),
               jnp.full((L,), r % L, jnp.int32)])
    t = jnp.max(tvec)
    v = plsc.load_gather(buf, [tvec])
    jb = t // CH

    def pre_body(j, c, buf=buf, v=v):
      base = j * CH
      out = []
      for u in range(U):
        x = buf[pl.ds(base + u * L, L)]
        out.append(c[u] + (x >= v).astype(jnp.int32))
      return tuple(out)

    c = lax.fori_loop(0, jb, pre_body,
                      tuple(jnp.zeros((L,), jnp.int32) for _ in range(U)))

    def suf_body(j, c, buf=buf, v=v):
      base = j * CH
      out = []
      for u in range(U):
        x = buf[pl.ds(base + u * L, L)]
        out.append(c[u] + (x > v).astype(jnp.int32))
      return tuple(out)

    c = lax.fori_loop(jb + 1, NIT, suf_body, c)

    base = jb * CH
    rank = jnp.int32(0)
    for u in range(U):
      x = buf[pl.ds(base + u * L, L)]
      idx = iotas[u] + base
      bc = (x > v) | ((x == v) & (idx < tvec))
      rank = rank + jnp.sum(bc.astype(jnp.int32) + c[u])
    hits = hits + jnp.where(rank < TOPK, jnp.float32(1.0), jnp.float32(0.0))

  obuf[...] = jnp.full((L,), hits, jnp.float32)
  pltpu.sync_copy(obuf, out_hbm.at[wid])


def _tc_merge(p_ref, o_ref):
  total = jnp.sum(p_ref[...]) * (1.0 / L)
  o_ref[...] = jnp.full((1, 1), (1.0 - total / NROWS) * 100.0, jnp.float32)


@jax.jit
def kernel(yhat, y):
  y2d = jnp.reshape(y, (NROWS, N))
  targets = pl.pallas_call(
      _tc_argmax,
      grid=(NBLK,),
      in_specs=[pl.BlockSpec((NROWS, CBLK), lambda j: (0, j))],
      out_specs=pl.BlockSpec((NROWS // L, L), lambda j: (0, 0)),
      out_shape=jax.ShapeDtypeStruct((NROWS // L, L), jnp.int32),
      scratch_shapes=[
          pltpu.VMEM((NROWS, 1), jnp.float32),
          pltpu.VMEM((NROWS, 1), jnp.int32),
      ],
  )(y2d)

  mesh = plsc.VectorSubcoreMesh(core_axis_name="c", subcore_axis_name="s")
  sc_k = functools.partial(
      pl.kernel,
      mesh=mesh,
      compiler_params=pltpu.CompilerParams(needs_layout_passes=False),
      out_type=jax.ShapeDtypeStruct((NW, L), jnp.float32),
      scratch_types=[
          pltpu.VMEM((N,), jnp.float32),
          pltpu.VMEM((N,), jnp.float32),
          pltpu.VMEM((NROWS // L, L), jnp.int32),
          pltpu.VMEM((L,), jnp.float32),
          pltpu.SemaphoreType.DMA,
          pltpu.SemaphoreType.DMA,
          pltpu.SemaphoreType.DMA,
      ],
  )(_sc_body)
  partial_hits = sc_k(yhat, targets)

  err = pl.pallas_call(
      _tc_merge,
      out_shape=jax.ShapeDtypeStruct((1, 1), jnp.float32),
  )(partial_hits)
  return jnp.reshape(err, ())
